# baseline (speedup 1.0000x reference)
import functools

import jax
import jax.numpy as jnp
from jax import lax
from jax.experimental import pallas as pl
from jax.experimental.pallas import tpu as pltpu
from jax.experimental.pallas import tpu_sc as plsc

_L = 64
_D = 2048
_B = 4
_S = 2048
_NC = 2
_NS = 16
_NW = _NC * _NS
_WPB = _NW // _B
_ROWS = (_L + _S) // _WPB


def _body(emb_hbm, prompt_hbm, out_hbm):
    wid = lax.axis_index("s") * _NC + lax.axis_index("c")
    b = wid // _WPB
    j = wid % _WPB

    @pl.when(j == 0)
    def _prompt_and_head():
        pltpu.sync_copy(prompt_hbm, out_hbm.at[b, pl.ds(0, _L), :])
        pltpu.sync_copy(
            emb_hbm.at[b, pl.ds(0, _ROWS - _L), :],
            out_hbm.at[b, pl.ds(_L, _ROWS - _L), :],
        )

    @pl.when(j > 0)
    def _tail():
        dst = j * _ROWS
        pltpu.sync_copy(
            emb_hbm.at[b, pl.ds(dst - _L, _ROWS), :],
            out_hbm.at[b, pl.ds(dst, _ROWS), :],
        )


@jax.jit
def kernel(embedded_input, prompt_table):
    out = pl.kernel(
        _body,
        out_type=jax.ShapeDtypeStruct((_B, _L + _S, _D), jnp.float32),
        mesh=plsc.VectorSubcoreMesh(core_axis_name="c", subcore_axis_name="s"),
    )(embedded_input, prompt_table)
    mask = jnp.ones((_B, _L), dtype=jnp.float32)
    return (out, mask)


# speedup vs baseline: 1.0075x; 1.0075x over previous
import jax
import jax.numpy as jnp
from jax.experimental import pallas as pl
from jax.experimental.pallas import tpu as pltpu

_L = 64
_D = 2048
_B = 4
_S = 2048


def _body(emb_ref, prompt_ref, out_ref, sem):
    copies = []
    for b in range(_B):
        copies.append(
            pltpu.make_async_copy(
                emb_ref.at[b], out_ref.at[b, pl.ds(_L, _S), :], sem
            )
        )
        copies.append(
            pltpu.make_async_copy(
                prompt_ref, out_ref.at[b, pl.ds(0, _L), :], sem
            )
        )
    for c in copies:
        c.start()
    for c in copies:
        c.wait()


@jax.jit
def kernel(embedded_input, prompt_table):
    out = pl.pallas_call(
        _body,
        out_shape=jax.ShapeDtypeStruct((_B, _L + _S, _D), jnp.float32),
        in_specs=[
            pl.BlockSpec(memory_space=pltpu.MemorySpace.HBM),
            pl.BlockSpec(memory_space=pltpu.MemorySpace.HBM),
        ],
        out_specs=pl.BlockSpec(memory_space=pltpu.MemorySpace.HBM),
        scratch_shapes=[pltpu.SemaphoreType.DMA],
    )(embedded_input, prompt_table)
    mask = jnp.ones((_B, _L), dtype=jnp.float32)
    return (out, mask)


# speedup vs baseline: 21.6074x; 21.4459x over previous
import jax
import jax.numpy as jnp
from jax.experimental import pallas as pl
from jax.experimental.pallas import tpu as pltpu

_L = 64
_D = 2048
_B = 4
_S = 2048


def _body(emb_ref, prompt_ref, out_ref):
    i = pl.program_id(1)

    @pl.when(i == 0)
    def _prompt():
        out_ref[0] = prompt_ref[...]

    @pl.when(i > 0)
    def _copy():
        out_ref[...] = emb_ref[...]


@jax.jit
def kernel(embedded_input, prompt_table):
    out = pl.pallas_call(
        _body,
        grid=(_B, 1 + _S // _L),
        in_specs=[
            pl.BlockSpec(
                (1, _L, _D),
                lambda b, i: (b, jnp.maximum(i - 1, 0), 0),
            ),
            pl.BlockSpec((_L, _D), lambda b, i: (0, 0)),
        ],
        out_specs=pl.BlockSpec((1, _L, _D), lambda b, i: (b, i, 0)),
        out_shape=jax.ShapeDtypeStruct((_B, _L + _S, _D), jnp.float32),
    )(embedded_input, prompt_table)
    mask = jnp.ones((_B, _L), dtype=jnp.float32)
    return (out, mask)


# speedup vs baseline: 48.1631x; 2.2290x over previous
import jax
import jax.numpy as jnp
from jax.experimental import pallas as pl
from jax.experimental.pallas import tpu as pltpu

_L = 64
_D = 2048
_B = 4
_S = 2048

_CHUNK = 256
_CPB = _S // _CHUNK
_NCHUNKS = _B * _CPB
_NBUF = 8
_LOOK = 4


def _chunk_src_dst(c, emb_ref, out_ref):
    b, j = divmod(c, _CPB)
    src = emb_ref.at[b, pl.ds(j * _CHUNK, _CHUNK), :]
    dst = out_ref.at[b, pl.ds(_L + j * _CHUNK, _CHUNK), :]
    return src, dst


def _body(emb_ref, prompt_ref, out_ref, buf, pbuf, in_sems, out_sems, psem):
    def in_dma(c):
        src, _ = _chunk_src_dst(c, emb_ref, out_ref)
        return pltpu.make_async_copy(src, buf.at[c % _NBUF], in_sems.at[c % _NBUF])

    def out_dma(c):
        _, dst = _chunk_src_dst(c, emb_ref, out_ref)
        return pltpu.make_async_copy(buf.at[c % _NBUF], dst, out_sems.at[c % _NBUF])

    pltpu.make_async_copy(prompt_ref, pbuf, psem).start()
    for c in range(_LOOK):
        in_dma(c).start()
    pltpu.make_async_copy(prompt_ref, pbuf, psem).wait()
    for b in range(_B):
        pltpu.make_async_copy(pbuf, out_ref.at[b, pl.ds(0, _L), :], psem).start()

    for c in range(_NCHUNKS):
        in_dma(c).wait()
        out_dma(c).start()
        nxt = c + _LOOK
        if nxt < _NCHUNKS:
            if nxt >= _NBUF:
                out_dma(nxt - _NBUF).wait()
            in_dma(nxt).start()

    for c in range(_NCHUNKS - _NBUF, _NCHUNKS):
        out_dma(c).wait()
    for b in range(_B):
        pltpu.make_async_copy(pbuf, out_ref.at[b, pl.ds(0, _L), :], psem).wait()


@jax.jit
def kernel(embedded_input, prompt_table):
    out = pl.pallas_call(
        _body,
        out_shape=jax.ShapeDtypeStruct((_B, _L + _S, _D), jnp.float32),
        in_specs=[
            pl.BlockSpec(memory_space=pltpu.MemorySpace.HBM),
            pl.BlockSpec(memory_space=pltpu.MemorySpace.HBM),
        ],
        out_specs=pl.BlockSpec(memory_space=pltpu.MemorySpace.HBM),
        scratch_shapes=[
            pltpu.VMEM((_NBUF, _CHUNK, _D), jnp.float32),
            pltpu.VMEM((_L, _D), jnp.float32),
            pltpu.SemaphoreType.DMA((_NBUF,)),
            pltpu.SemaphoreType.DMA((_NBUF,)),
            pltpu.SemaphoreType.DMA,
        ],
    )(embedded_input, prompt_table)
    mask = jnp.ones((_B, _L), dtype=jnp.float32)
    return (out, mask)


# speedup vs baseline: 48.6995x; 1.0111x over previous
import jax
import jax.numpy as jnp
from jax.experimental import pallas as pl
from jax.experimental.pallas import tpu as pltpu

_L = 64
_D = 2048
_B = 4
_S = 2048

_CHUNK = 512
_CPB = _S // _CHUNK
_NCHUNKS = _B * _CPB
_NBUF = 4
_LOOK = 2


def _chunk_src_dst(c, emb_ref, out_ref):
    b, j = divmod(c, _CPB)
    src = emb_ref.at[b, pl.ds(j * _CHUNK, _CHUNK), :]
    dst = out_ref.at[b, pl.ds(_L + j * _CHUNK, _CHUNK), :]
    return src, dst


def _body(emb_ref, prompt_ref, out_ref, buf, pbuf, in_sems, out_sems, psem):
    def in_dma(c):
        src, _ = _chunk_src_dst(c, emb_ref, out_ref)
        return pltpu.make_async_copy(src, buf.at[c % _NBUF], in_sems.at[c % _NBUF])

    def out_dma(c):
        _, dst = _chunk_src_dst(c, emb_ref, out_ref)
        return pltpu.make_async_copy(buf.at[c % _NBUF], dst, out_sems.at[c % _NBUF])

    pltpu.make_async_copy(prompt_ref, pbuf, psem).start()
    for c in range(_LOOK):
        in_dma(c).start()
    pltpu.make_async_copy(prompt_ref, pbuf, psem).wait()
    for b in range(_B):
        pltpu.make_async_copy(pbuf, out_ref.at[b, pl.ds(0, _L), :], psem).start()

    for c in range(_NCHUNKS):
        in_dma(c).wait()
        out_dma(c).start()
        nxt = c + _LOOK
        if nxt < _NCHUNKS:
            if nxt >= _NBUF:
                out_dma(nxt - _NBUF).wait()
            in_dma(nxt).start()

    for c in range(_NCHUNKS - _NBUF, _NCHUNKS):
        out_dma(c).wait()
    for b in range(_B):
        pltpu.make_async_copy(pbuf, out_ref.at[b, pl.ds(0, _L), :], psem).wait()


@jax.jit
def kernel(embedded_input, prompt_table):
    out = pl.pallas_call(
        _body,
        out_shape=jax.ShapeDtypeStruct((_B, _L + _S, _D), jnp.float32),
        in_specs=[
            pl.BlockSpec(memory_space=pltpu.MemorySpace.HBM),
            pl.BlockSpec(memory_space=pltpu.MemorySpace.HBM),
        ],
        out_specs=pl.BlockSpec(memory_space=pltpu.MemorySpace.HBM),
        scratch_shapes=[
            pltpu.VMEM((_NBUF, _CHUNK, _D), jnp.float32),
            pltpu.VMEM((_L, _D), jnp.float32),
            pltpu.SemaphoreType.DMA((_NBUF,)),
            pltpu.SemaphoreType.DMA((_NBUF,)),
            pltpu.SemaphoreType.DMA,
        ],
    )(embedded_input, prompt_table)
    mask = jnp.ones((_B, _L), dtype=jnp.float32)
    return (out, mask)


# speedup vs baseline: 49.3196x; 1.0127x over previous
import jax
import jax.numpy as jnp
from jax.experimental import pallas as pl
from jax.experimental.pallas import tpu as pltpu

_L = 64
_D = 2048
_B = 4
_S = 2048

_CHUNK = 1024
_CPB = _S // _CHUNK
_NCHUNKS = _B * _CPB
_NBUF = 4
_LOOK = 2


def _chunk_src_dst(c, emb_ref, out_ref):
    b, j = divmod(c, _CPB)
    src = emb_ref.at[b, pl.ds(j * _CHUNK, _CHUNK), :]
    dst = out_ref.at[b, pl.ds(_L + j * _CHUNK, _CHUNK), :]
    return src, dst


def _body(emb_ref, prompt_ref, out_ref, buf, pbuf, in_sems, out_sems, psem):
    def in_dma(c):
        src, _ = _chunk_src_dst(c, emb_ref, out_ref)
        return pltpu.make_async_copy(src, buf.at[c % _NBUF], in_sems.at[c % _NBUF])

    def out_dma(c):
        _, dst = _chunk_src_dst(c, emb_ref, out_ref)
        return pltpu.make_async_copy(buf.at[c % _NBUF], dst, out_sems.at[c % _NBUF])

    pltpu.make_async_copy(prompt_ref, pbuf, psem).start()
    for c in range(_LOOK):
        in_dma(c).start()
    pltpu.make_async_copy(prompt_ref, pbuf, psem).wait()
    for b in range(_B):
        pltpu.make_async_copy(pbuf, out_ref.at[b, pl.ds(0, _L), :], psem).start()

    for c in range(_NCHUNKS):
        in_dma(c).wait()
        out_dma(c).start()
        nxt = c + _LOOK
        if nxt < _NCHUNKS:
            if nxt >= _NBUF:
                out_dma(nxt - _NBUF).wait()
            in_dma(nxt).start()

    for c in range(_NCHUNKS - _NBUF, _NCHUNKS):
        out_dma(c).wait()
    for b in range(_B):
        pltpu.make_async_copy(pbuf, out_ref.at[b, pl.ds(0, _L), :], psem).wait()


@jax.jit
def kernel(embedded_input, prompt_table):
    out = pl.pallas_call(
        _body,
        out_shape=jax.ShapeDtypeStruct((_B, _L + _S, _D), jnp.float32),
        in_specs=[
            pl.BlockSpec(memory_space=pltpu.MemorySpace.HBM),
            pl.BlockSpec(memory_space=pltpu.MemorySpace.HBM),
        ],
        out_specs=pl.BlockSpec(memory_space=pltpu.MemorySpace.HBM),
        scratch_shapes=[
            pltpu.VMEM((_NBUF, _CHUNK, _D), jnp.float32),
            pltpu.VMEM((_L, _D), jnp.float32),
            pltpu.SemaphoreType.DMA((_NBUF,)),
            pltpu.SemaphoreType.DMA((_NBUF,)),
            pltpu.SemaphoreType.DMA,
        ],
    )(embedded_input, prompt_table)
    mask = jnp.ones((_B, _L), dtype=jnp.float32)
    return (out, mask)
